# baseline (speedup 1.0000x reference)
import functools

import jax
import jax.numpy as jnp
import numpy as np
from jax import lax
from jax.experimental import pallas as pl
from jax.experimental.pallas import tpu as pltpu
from jax.experimental.pallas import tpu_sc as plsc

K_TOP = 32
FEAT_BLK = 1024
NEG_BIG = np.float32(-3.0e38)
MARGIN = np.float32(0.1)


def _mm_body(x_ref, w_ref, b_ref, y_ref):
    y = lax.dot_general(x_ref[0], w_ref[...], (((1,), (1,)), ((), ())),
                        preferred_element_type=jnp.float32)
    y_ref[...] = y + b_ref[...][None, :]


def _matmul(x, W, b):
    B, S, D = x.shape
    F = W.shape[0]
    grid = (F // FEAT_BLK,)
    return pl.pallas_call(
        _mm_body,
        grid=grid,
        in_specs=[
            pl.BlockSpec((1, S, D), lambda f: (0, 0, 0)),
            pl.BlockSpec((FEAT_BLK, D), lambda f: (f, 0)),
            pl.BlockSpec((FEAT_BLK,), lambda f: (f,)),
        ],
        out_specs=pl.BlockSpec((S, FEAT_BLK), lambda f: (0, f)),
        out_shape=jax.ShapeDtypeStruct((S, F), jnp.float32),
        compiler_params=pltpu.CompilerParams(
            dimension_semantics=("arbitrary",)),
    )(x, W, b)


def _vec(x):
    return lax.broadcast_in_dim(x, (16,), ())


def _merge32(sv0, si0, sv1, si1, nv, ni):
    nvs, nis = plsc.sort_key_val(nv, ni)
    rv = lax.rev(nvs, (0,))
    ri = lax.rev(nis, (0,))
    sel = sv0 >= rv
    hv = jnp.where(sel, sv0, rv)
    hi = jnp.where(sel, si0, ri)
    hvs, his = plsc.sort_key_val(hv, hi)
    rhv = lax.rev(hvs, (0,))
    rhi = lax.rev(his, (0,))
    sel2 = sv1 >= rhv
    new_hi_v = jnp.where(sel2, sv1, rhv)
    new_hi_i = jnp.where(sel2, si1, rhi)
    new_lo_v = jnp.where(sel2, rhv, sv1)
    new_lo_i = jnp.where(sel2, rhi, si1)
    lv, li = plsc.sort_key_val(new_lo_v, new_lo_i)
    uv, ui = plsc.sort_key_val(new_hi_v, new_hi_i)
    return lv, li, uv, ui


def _sc_select(yf, S, F, interpret=False):
    mesh = plsc.VectorSubcoreMesh(core_axis_name="c", subcore_axis_name="s",
                                  num_cores=2, num_subcores=16)
    nrows = S // 32

    @functools.partial(
        pl.kernel,
        out_type=jax.ShapeDtypeStruct((S * F,), jnp.float32),
        mesh=mesh,
        interpret=interpret,
        compiler_params=pltpu.CompilerParams(needs_layout_passes=False),
        scratch_types=[
            pltpu.VMEM((2 * F,), jnp.float32),
            pltpu.VMEM((2 * F,), jnp.float32),
            pltpu.VMEM((F + 16,), jnp.float32),
            pltpu.VMEM((F + 16,), jnp.int32),
            pltpu.VMEM((64,), jnp.int32),
            pltpu.VMEM((16,), jnp.float32),
            pltpu.VMEM((32,), jnp.float32),
            pltpu.VMEM((32,), jnp.int32),
            pltpu.SemaphoreType.DMA,
            pltpu.SemaphoreType.DMA,
            pltpu.SemaphoreType.DMA,
            pltpu.SemaphoreType.DMA,
        ],
    )
    def sel(y_hbm, out_hbm, ybuf, zrow, cv, ci, sidx, tscr, stv, sti,
            in0, in1, ot0, ot1):
        wid = lax.axis_index("s") * 2 + lax.axis_index("c")
        base = wid * nrows
        iota = lax.iota(jnp.int32, 16)
        zeros16 = jnp.zeros((16,), jnp.float32)
        insem = (in0, in1)
        otsem = (ot0, ot1)

        def _z(i, _):
            zrow[pl.ds(i * 16, 16)] = zeros16
            zrow[pl.ds(F + i * 16, 16)] = zeros16
            return _

        lax.fori_loop(0, F // 16, _z, 0)
        tscr[pl.ds(0, 16)] = _vec(NEG_BIG)

        pltpu.async_copy(y_hbm.at[pl.ds(base * F, F)],
                         ybuf.at[pl.ds(0, F)], in0)

        def popcnt(m):
            return plsc.all_reduce_population_count(m)[0]

        def collect(bo, tcv, iv):
            def grp(gg, L):
                offs = gg * 128

                def one(j, L2):
                    v = ybuf[pl.ds(bo + offs + j * 16, 16)]
                    m = v >= tcv
                    c = popcnt(m)

                    def hit(Lh):
                        plsc.store_compressed(cv.at[pl.ds(Lh, 16)], v,
                                              mask=m)
                        plsc.store_compressed(
                            ci.at[pl.ds(Lh, 16)],
                            iota + (offs + j * 16), mask=m)
                        return Lh + c

                    return lax.cond(c > 0, hit, lambda Lh: Lh, L2)

                om = ybuf[pl.ds(bo + offs, 16)] >= tcv
                for j in range(1, 8):
                    om = jnp.logical_or(
                        om, ybuf[pl.ds(bo + offs + j * 16, 16)] >= tcv)
                anyc = popcnt(om)

                def ghit(Lg):
                    return lax.fori_loop(0, 8, one, Lg)

                return lax.cond(anyc > 0, ghit, lambda Lg: Lg, L)

            return lax.fori_loop(0, F // 128, grp, iv)

        def process_row(i, g, buf):
            bo = buf * F
            pltpu.make_async_copy(y_hbm.at[pl.ds(g * F, F)],
                                  ybuf.at[pl.ds(bo, F)],
                                  insem[buf]).wait()

            @pl.when(i == 0)
            def _boot():
                if buf == 0:
                    def mx(t, _):
                        mv = jnp.maximum(ybuf[pl.ds(t * 64, 16)],
                                         ybuf[pl.ds(t * 64 + 16, 16)])
                        mv2 = jnp.maximum(ybuf[pl.ds(t * 64 + 32, 16)],
                                          ybuf[pl.ds(t * 64 + 48, 16)])
                        cur = tscr[pl.ds(0, 16)]
                        tscr[pl.ds(0, 16)] = jnp.maximum(
                            cur, jnp.maximum(mv, mv2))
                        return _

                    lax.fori_loop(0, F // 64, mx, 0)
                    m16 = tscr[pl.ds(0, 16)]
                    sm, _d = plsc.sort_key_val(m16, iota)
                    tscr[pl.ds(0, 16)] = _vec(sm[15] * np.float32(0.6))

            tcv = tscr[pl.ds(0, 16)]
            L = collect(bo, tcv, 0)
            L = lax.cond(L < K_TOP,
                         lambda _: collect(bo, _vec(NEG_BIG), 0),
                         lambda Lk: Lk, L)
            cv[pl.ds(L, 16)] = _vec(NEG_BIG)
            ci[pl.ds(L, 16)] = iota

            stv[pl.ds(0, 16)] = _vec(NEG_BIG)
            stv[pl.ds(16, 16)] = _vec(NEG_BIG)
            sti[pl.ds(0, 16)] = iota
            sti[pl.ds(16, 16)] = iota + 16
            nvec = (L + 15) // 16

            def mstep(t, d):
                nv = cv[pl.ds(t * 16, 16)]
                ni = ci[pl.ds(t * 16, 16)]
                a, b_, c_, d_ = _merge32(
                    stv[pl.ds(0, 16)], sti[pl.ds(0, 16)],
                    stv[pl.ds(16, 16)], sti[pl.ds(16, 16)], nv, ni)
                stv[pl.ds(0, 16)] = a
                sti[pl.ds(0, 16)] = b_
                stv[pl.ds(16, 16)] = c_
                sti[pl.ds(16, 16)] = d_
                return d

            lax.fori_loop(0, nvec, mstep, 0)
            sv0 = stv[pl.ds(0, 16)]
            sv1 = stv[pl.ds(16, 16)]
            si0 = sti[pl.ds(0, 16)] + bo
            si1 = sti[pl.ds(16, 16)] + bo

            @pl.when(i >= 1)
            def _drain():
                pltpu.make_async_copy(zrow.at[pl.ds(bo, F)],
                                      out_hbm.at[pl.ds((g - 2) * F, F)],
                                      otsem[buf]).wait()
                oi0 = sidx[pl.ds(buf * 32, 16)]
                oi1 = sidx[pl.ds(buf * 32 + 16, 16)]
                plsc.store_scatter(zrow, [oi0], zeros16)
                plsc.store_scatter(zrow, [oi1], zeros16)

            plsc.store_scatter(zrow, [si0], jnp.maximum(sv0, 0.0))
            plsc.store_scatter(zrow, [si1], jnp.maximum(sv1, 0.0))
            sidx[pl.ds(buf * 32, 16)] = si0
            sidx[pl.ds(buf * 32 + 16, 16)] = si1
            pltpu.async_copy(zrow.at[pl.ds(bo, F)],
                             out_hbm.at[pl.ds(g * F, F)], otsem[buf])

            tscr[pl.ds(0, 16)] = _vec(sv0[0] - MARGIN)

        def pair_step(i, d):
            g0 = base + 2 * i
            pltpu.async_copy(y_hbm.at[pl.ds((g0 + 1) * F, F)],
                             ybuf.at[pl.ds(F, F)], in1)
            process_row(i, g0, 0)

            @pl.when(i < nrows // 2 - 1)
            def _pref():
                pltpu.async_copy(y_hbm.at[pl.ds((g0 + 2) * F, F)],
                                 ybuf.at[pl.ds(0, F)], in0)

            process_row(i, g0 + 1, 1)
            return d

        lax.fori_loop(0, nrows // 2, pair_step, 0)
        pltpu.make_async_copy(zrow.at[pl.ds(0, F)],
                              out_hbm.at[pl.ds((base + nrows - 2) * F, F)],
                              ot0).wait()
        pltpu.make_async_copy(zrow.at[pl.ds(F, F)],
                              out_hbm.at[pl.ds((base + nrows - 1) * F, F)],
                              ot1).wait()

    return sel(yf)


@jax.jit
def _run(x, W, b):
    B, S, D = x.shape
    F = W.shape[0]
    y = _matmul(x.astype(jnp.bfloat16), W.astype(jnp.bfloat16), b)
    out = _sc_select(y.reshape(S * F), S, F)
    return out.reshape(B, S, F)


def kernel(x, token_mask, W, b):
    del token_mask
    return _run(x, W, b)


# speedup vs baseline: 1.4414x; 1.4414x over previous
import functools

import jax
import jax.numpy as jnp
import numpy as np
from jax import lax
from jax.experimental import pallas as pl
from jax.experimental.pallas import tpu as pltpu
from jax.experimental.pallas import tpu_sc as plsc

K_TOP = 32
FEAT_BLK = 1024
NEG_BIG = np.float32(-3.0e38)
MARGIN = np.float32(0.25)


def _mm_body(x_ref, w_ref, b_ref, y_ref):
    y = lax.dot_general(x_ref[0], w_ref[...], (((1,), (1,)), ((), ())),
                        preferred_element_type=jnp.float32)
    y_ref[...] = y + b_ref[...][None, :]


def _matmul(x, W, b):
    B, S, D = x.shape
    F = W.shape[0]
    grid = (F // FEAT_BLK,)
    return pl.pallas_call(
        _mm_body,
        grid=grid,
        in_specs=[
            pl.BlockSpec((1, S, D), lambda f: (0, 0, 0)),
            pl.BlockSpec((FEAT_BLK, D), lambda f: (f, 0)),
            pl.BlockSpec((FEAT_BLK,), lambda f: (f,)),
        ],
        out_specs=pl.BlockSpec((S, FEAT_BLK), lambda f: (0, f)),
        out_shape=jax.ShapeDtypeStruct((S, F), jnp.float32),
        compiler_params=pltpu.CompilerParams(
            dimension_semantics=("arbitrary",)),
    )(x, W, b)


def _vec(x):
    return lax.broadcast_in_dim(x, (16,), ())


def _merge32(sv0, si0, sv1, si1, nv, ni):
    nvs, nis = plsc.sort_key_val(nv, ni)
    rv = lax.rev(nvs, (0,))
    ri = lax.rev(nis, (0,))
    sel = sv0 >= rv
    hv = jnp.where(sel, sv0, rv)
    hi = jnp.where(sel, si0, ri)
    hvs, his = plsc.sort_key_val(hv, hi)
    rhv = lax.rev(hvs, (0,))
    rhi = lax.rev(his, (0,))
    sel2 = sv1 >= rhv
    new_hi_v = jnp.where(sel2, sv1, rhv)
    new_hi_i = jnp.where(sel2, si1, rhi)
    new_lo_v = jnp.where(sel2, rhv, sv1)
    new_lo_i = jnp.where(sel2, rhi, si1)
    lv, li = plsc.sort_key_val(new_lo_v, new_lo_i)
    uv, ui = plsc.sort_key_val(new_hi_v, new_hi_i)
    return lv, li, uv, ui


def _sc_select(yf, S, F, interpret=False):
    mesh = plsc.VectorSubcoreMesh(core_axis_name="c", subcore_axis_name="s",
                                  num_cores=2, num_subcores=16)
    nrows = S // 32

    @functools.partial(
        pl.kernel,
        out_type=jax.ShapeDtypeStruct((S * F,), jnp.float32),
        mesh=mesh,
        interpret=interpret,
        compiler_params=pltpu.CompilerParams(needs_layout_passes=False),
        scratch_types=[
            pltpu.VMEM((2 * F,), jnp.float32),
            pltpu.VMEM((2 * F,), jnp.float32),
            pltpu.VMEM((F + 16,), jnp.float32),
            pltpu.VMEM((F + 16,), jnp.int32),
            pltpu.VMEM((64,), jnp.int32),
            pltpu.VMEM((16,), jnp.float32),
            pltpu.VMEM((32,), jnp.float32),
            pltpu.VMEM((32,), jnp.int32),
            pltpu.SemaphoreType.DMA,
            pltpu.SemaphoreType.DMA,
            pltpu.SemaphoreType.DMA,
            pltpu.SemaphoreType.DMA,
        ],
    )
    def sel(y_hbm, out_hbm, ybuf, zrow, cv, ci, sidx, tscr, stv, sti,
            in0, in1, ot0, ot1):
        wid = lax.axis_index("s") * 2 + lax.axis_index("c")
        base = wid * nrows
        iota = lax.iota(jnp.int32, 16)
        zeros16 = jnp.zeros((16,), jnp.float32)
        insem = (in0, in1)
        otsem = (ot0, ot1)

        def _z(i, _):
            zrow[pl.ds(i * 16, 16)] = zeros16
            zrow[pl.ds(F + i * 16, 16)] = zeros16
            return _

        lax.fori_loop(0, F // 16, _z, 0)
        tscr[pl.ds(0, 16)] = _vec(NEG_BIG)

        pltpu.async_copy(y_hbm.at[pl.ds(base * F, F)],
                         ybuf.at[pl.ds(0, F)], in0)

        def popcnt(m):
            return plsc.all_reduce_population_count(m)[0]

        def collect(bo, tcv, iv):
            def grp(gg, L):
                offs = gg * 128

                def one(j, L2):
                    v = ybuf[pl.ds(bo + offs + j * 16, 16)]
                    m = v >= tcv
                    c = popcnt(m)

                    def hit(Lh):
                        plsc.store_compressed(cv.at[pl.ds(Lh, 16)], v,
                                              mask=m)
                        plsc.store_compressed(
                            ci.at[pl.ds(Lh, 16)],
                            iota + (offs + j * 16), mask=m)
                        return Lh + c

                    return lax.cond(c > 0, hit, lambda Lh: Lh, L2)

                om = ybuf[pl.ds(bo + offs, 16)] >= tcv
                for j in range(1, 8):
                    om = jnp.logical_or(
                        om, ybuf[pl.ds(bo + offs + j * 16, 16)] >= tcv)
                anyc = popcnt(om)

                def ghit(Lg):
                    return lax.fori_loop(0, 8, one, Lg)

                return lax.cond(anyc > 0, ghit, lambda Lg: Lg, L)

            return lax.fori_loop(0, F // 128, grp, iv)

        def process_row(i, g, buf):
            bo = buf * F
            pltpu.make_async_copy(y_hbm.at[pl.ds(g * F, F)],
                                  ybuf.at[pl.ds(bo, F)],
                                  insem[buf]).wait()

            @pl.when(i == 0)
            def _boot():
                if buf == 0:
                    def mx(t, _):
                        mv = jnp.maximum(ybuf[pl.ds(t * 64, 16)],
                                         ybuf[pl.ds(t * 64 + 16, 16)])
                        mv2 = jnp.maximum(ybuf[pl.ds(t * 64 + 32, 16)],
                                          ybuf[pl.ds(t * 64 + 48, 16)])
                        cur = tscr[pl.ds(0, 16)]
                        tscr[pl.ds(0, 16)] = jnp.maximum(
                            cur, jnp.maximum(mv, mv2))
                        return _

                    lax.fori_loop(0, F // 64, mx, 0)
                    m16 = tscr[pl.ds(0, 16)]
                    sm, _d = plsc.sort_key_val(m16, iota)
                    tscr[pl.ds(0, 16)] = _vec(sm[15] * np.float32(0.6))

            tcv = tscr[pl.ds(0, 16)]
            L = collect(bo, tcv, 0)
            L = lax.cond(L < K_TOP,
                         lambda _: collect(bo, _vec(NEG_BIG), 0),
                         lambda Lk: Lk, L)
            cv[pl.ds(L, 16)] = _vec(NEG_BIG)
            ci[pl.ds(L, 16)] = iota

            stv[pl.ds(0, 16)] = _vec(NEG_BIG)
            stv[pl.ds(16, 16)] = _vec(NEG_BIG)
            sti[pl.ds(0, 16)] = iota
            sti[pl.ds(16, 16)] = iota + 16
            nvec = (L + 15) // 16

            def mstep(t, d):
                nv = cv[pl.ds(t * 16, 16)]
                ni = ci[pl.ds(t * 16, 16)]
                a, b_, c_, d_ = _merge32(
                    stv[pl.ds(0, 16)], sti[pl.ds(0, 16)],
                    stv[pl.ds(16, 16)], sti[pl.ds(16, 16)], nv, ni)
                stv[pl.ds(0, 16)] = a
                sti[pl.ds(0, 16)] = b_
                stv[pl.ds(16, 16)] = c_
                sti[pl.ds(16, 16)] = d_
                return d

            lax.fori_loop(0, nvec, mstep, 0)
            sv0 = stv[pl.ds(0, 16)]
            sv1 = stv[pl.ds(16, 16)]
            si0 = sti[pl.ds(0, 16)] + bo
            si1 = sti[pl.ds(16, 16)] + bo

            @pl.when(i >= 1)
            def _drain():
                pltpu.make_async_copy(zrow.at[pl.ds(bo, F)],
                                      out_hbm.at[pl.ds((g - 2) * F, F)],
                                      otsem[buf]).wait()
                oi0 = sidx[pl.ds(buf * 32, 16)]
                oi1 = sidx[pl.ds(buf * 32 + 16, 16)]
                plsc.store_scatter(zrow, [oi0], zeros16)
                plsc.store_scatter(zrow, [oi1], zeros16)

            plsc.store_scatter(zrow, [si0], jnp.maximum(sv0, 0.0))
            plsc.store_scatter(zrow, [si1], jnp.maximum(sv1, 0.0))
            sidx[pl.ds(buf * 32, 16)] = si0
            sidx[pl.ds(buf * 32 + 16, 16)] = si1
            pltpu.async_copy(zrow.at[pl.ds(bo, F)],
                             out_hbm.at[pl.ds(g * F, F)], otsem[buf])

            tscr[pl.ds(0, 16)] = _vec(sv0[0] - MARGIN)

        def pair_step(i, d):
            g0 = base + 2 * i
            pltpu.async_copy(y_hbm.at[pl.ds((g0 + 1) * F, F)],
                             ybuf.at[pl.ds(F, F)], in1)
            process_row(i, g0, 0)

            @pl.when(i < nrows // 2 - 1)
            def _pref():
                pltpu.async_copy(y_hbm.at[pl.ds((g0 + 2) * F, F)],
                                 ybuf.at[pl.ds(0, F)], in0)

            process_row(i, g0 + 1, 1)
            return d

        lax.fori_loop(0, nrows // 2, pair_step, 0)
        pltpu.make_async_copy(zrow.at[pl.ds(0, F)],
                              out_hbm.at[pl.ds((base + nrows - 2) * F, F)],
                              ot0).wait()
        pltpu.make_async_copy(zrow.at[pl.ds(F, F)],
                              out_hbm.at[pl.ds((base + nrows - 1) * F, F)],
                              ot1).wait()

    return sel(yf)


@jax.jit
def _run(x, W, b):
    B, S, D = x.shape
    F = W.shape[0]
    y = _matmul(x.astype(jnp.bfloat16), W.astype(jnp.bfloat16), b)
    out = _sc_select(y.reshape(S * F), S, F)
    return out.reshape(B, S, F)


def kernel(x, token_mask, W, b):
    del token_mask
    return _run(x, W, b)


# speedup vs baseline: 2.1069x; 1.4617x over previous
import functools

import jax
import jax.numpy as jnp
import numpy as np
from jax import lax
from jax.experimental import pallas as pl
from jax.experimental.pallas import tpu as pltpu
from jax.experimental.pallas import tpu_sc as plsc

K_TOP = 32
FEAT_BLK = 1024
NEG_BIG = np.float32(-3.0e38)
MARGIN = np.float32(0.25)


def _mm_body(x_ref, w_ref, b_ref, y_ref):
    y = lax.dot_general(x_ref[0], w_ref[...], (((1,), (1,)), ((), ())),
                        preferred_element_type=jnp.float32)
    y_ref[...] = y + b_ref[...][None, :]


def _matmul(x, W, b):
    B, S, D = x.shape
    F = W.shape[0]
    grid = (F // FEAT_BLK,)
    return pl.pallas_call(
        _mm_body,
        grid=grid,
        in_specs=[
            pl.BlockSpec((1, S, D), lambda f: (0, 0, 0)),
            pl.BlockSpec((FEAT_BLK, D), lambda f: (f, 0)),
            pl.BlockSpec((FEAT_BLK,), lambda f: (f,)),
        ],
        out_specs=pl.BlockSpec((S, FEAT_BLK), lambda f: (0, f)),
        out_shape=jax.ShapeDtypeStruct((S, F), jnp.float32),
        compiler_params=pltpu.CompilerParams(
            dimension_semantics=("arbitrary",)),
    )(x, W, b)


def _vec(x):
    return lax.broadcast_in_dim(x, (16,), ())


def _merge32(sv0, si0, sv1, si1, nv, ni):
    nvs, nis = plsc.sort_key_val(nv, ni)
    rv = lax.rev(nvs, (0,))
    ri = lax.rev(nis, (0,))
    sel = sv0 >= rv
    hv = jnp.where(sel, sv0, rv)
    hi = jnp.where(sel, si0, ri)
    hvs, his = plsc.sort_key_val(hv, hi)
    rhv = lax.rev(hvs, (0,))
    rhi = lax.rev(his, (0,))
    sel2 = sv1 >= rhv
    new_hi_v = jnp.where(sel2, sv1, rhv)
    new_hi_i = jnp.where(sel2, si1, rhi)
    new_lo_v = jnp.where(sel2, rhv, sv1)
    new_lo_i = jnp.where(sel2, rhi, si1)
    lv, li = plsc.sort_key_val(new_lo_v, new_lo_i)
    uv, ui = plsc.sort_key_val(new_hi_v, new_hi_i)
    return lv, li, uv, ui


def _sc_select(yf, S, F, interpret=False):
    mesh = plsc.VectorSubcoreMesh(core_axis_name="c", subcore_axis_name="s",
                                  num_cores=2, num_subcores=16)
    nrows = S // 32

    @functools.partial(
        pl.kernel,
        out_type=jax.ShapeDtypeStruct((S * F,), jnp.float32),
        mesh=mesh,
        interpret=interpret,
        compiler_params=pltpu.CompilerParams(needs_layout_passes=False),
        scratch_types=[
            pltpu.VMEM((2 * F,), jnp.float32),
            pltpu.VMEM((2 * F,), jnp.float32),
            pltpu.VMEM((F + 16,), jnp.float32),
            pltpu.VMEM((F + 16,), jnp.int32),
            pltpu.VMEM((64,), jnp.int32),
            pltpu.VMEM((16,), jnp.float32),
            pltpu.VMEM((32,), jnp.float32),
            pltpu.VMEM((32,), jnp.int32),
            pltpu.SemaphoreType.DMA,
            pltpu.SemaphoreType.DMA,
            pltpu.SemaphoreType.DMA,
            pltpu.SemaphoreType.DMA,
        ],
    )
    def sel(y_hbm, out_hbm, ybuf, zrow, cv, ci, sidx, tscr, stv, sti,
            in0, in1, ot0, ot1):
        wid = lax.axis_index("s") * 2 + lax.axis_index("c")
        base = wid * nrows
        iota = lax.iota(jnp.int32, 16)
        zeros16 = jnp.zeros((16,), jnp.float32)
        insem = (in0, in1)
        otsem = (ot0, ot1)

        def _z(i, _):
            zrow[pl.ds(i * 16, 16)] = zeros16
            zrow[pl.ds(F + i * 16, 16)] = zeros16
            return _

        lax.fori_loop(0, F // 16, _z, 0)
        tscr[pl.ds(0, 16)] = _vec(NEG_BIG)

        pltpu.async_copy(y_hbm.at[pl.ds(base * F, F)],
                         ybuf.at[pl.ds(0, F)], in0)

        def popcnt(m):
            return plsc.all_reduce_population_count(m)[0]

        def collect(bo, tcv, iv):
            def grp(gg, L):
                offs = gg * 128

                om = ybuf[pl.ds(bo + offs, 16)] >= tcv
                for j in range(1, 8):
                    om = jnp.logical_or(
                        om, ybuf[pl.ds(bo + offs + j * 16, 16)] >= tcv)
                anyc = popcnt(om)

                def ghit(Lg):
                    ms = []
                    cvec = jnp.zeros((16,), jnp.int32)
                    for j in range(8):
                        v = ybuf[pl.ds(bo + offs + j * 16, 16)]
                        m = v >= tcv
                        ms.append((v, m))
                        pc = plsc.all_reduce_population_count(m)
                        cvec = jnp.where(iota == j, pc, cvec)
                    pref = plsc.cumsum(cvec)
                    excl = pref - cvec
                    for j in range(8):
                        v, m = ms[j]
                        o = Lg + excl[j]
                        plsc.store_compressed(cv.at[pl.ds(o, 16)], v,
                                              mask=m)
                        plsc.store_compressed(
                            ci.at[pl.ds(o, 16)],
                            iota + (offs + j * 16), mask=m)
                    return Lg + pref[15]

                return lax.cond(anyc > 0, ghit, lambda Lg: Lg, L)

            return lax.fori_loop(0, F // 128, grp, iv)

        def process_row(i, g, buf):
            bo = buf * F
            pltpu.make_async_copy(y_hbm.at[pl.ds(g * F, F)],
                                  ybuf.at[pl.ds(bo, F)],
                                  insem[buf]).wait()

            @pl.when(i == 0)
            def _boot():
                if buf == 0:
                    def mx(t, _):
                        mv = jnp.maximum(ybuf[pl.ds(t * 64, 16)],
                                         ybuf[pl.ds(t * 64 + 16, 16)])
                        mv2 = jnp.maximum(ybuf[pl.ds(t * 64 + 32, 16)],
                                          ybuf[pl.ds(t * 64 + 48, 16)])
                        cur = tscr[pl.ds(0, 16)]
                        tscr[pl.ds(0, 16)] = jnp.maximum(
                            cur, jnp.maximum(mv, mv2))
                        return _

                    lax.fori_loop(0, F // 64, mx, 0)
                    m16 = tscr[pl.ds(0, 16)]
                    sm, _d = plsc.sort_key_val(m16, iota)
                    tscr[pl.ds(0, 16)] = _vec(sm[15] * np.float32(0.6))

            tcv = tscr[pl.ds(0, 16)]
            L = collect(bo, tcv, 0)
            L = lax.cond(L < K_TOP,
                         lambda _: collect(bo, _vec(NEG_BIG), 0),
                         lambda Lk: Lk, L)
            cv[pl.ds(L, 16)] = _vec(NEG_BIG)
            ci[pl.ds(L, 16)] = iota

            stv[pl.ds(0, 16)] = _vec(NEG_BIG)
            stv[pl.ds(16, 16)] = _vec(NEG_BIG)
            sti[pl.ds(0, 16)] = iota
            sti[pl.ds(16, 16)] = iota + 16
            nvec = (L + 15) // 16

            def mstep(t, d):
                nv = cv[pl.ds(t * 16, 16)]
                ni = ci[pl.ds(t * 16, 16)]
                a, b_, c_, d_ = _merge32(
                    stv[pl.ds(0, 16)], sti[pl.ds(0, 16)],
                    stv[pl.ds(16, 16)], sti[pl.ds(16, 16)], nv, ni)
                stv[pl.ds(0, 16)] = a
                sti[pl.ds(0, 16)] = b_
                stv[pl.ds(16, 16)] = c_
                sti[pl.ds(16, 16)] = d_
                return d

            lax.fori_loop(0, nvec, mstep, 0)
            sv0 = stv[pl.ds(0, 16)]
            sv1 = stv[pl.ds(16, 16)]
            si0 = sti[pl.ds(0, 16)] + bo
            si1 = sti[pl.ds(16, 16)] + bo

            @pl.when(i >= 1)
            def _drain():
                pltpu.make_async_copy(zrow.at[pl.ds(bo, F)],
                                      out_hbm.at[pl.ds((g - 2) * F, F)],
                                      otsem[buf]).wait()
                oi0 = sidx[pl.ds(buf * 32, 16)]
                oi1 = sidx[pl.ds(buf * 32 + 16, 16)]
                plsc.store_scatter(zrow, [oi0], zeros16)
                plsc.store_scatter(zrow, [oi1], zeros16)

            plsc.store_scatter(zrow, [si0], jnp.maximum(sv0, 0.0))
            plsc.store_scatter(zrow, [si1], jnp.maximum(sv1, 0.0))
            sidx[pl.ds(buf * 32, 16)] = si0
            sidx[pl.ds(buf * 32 + 16, 16)] = si1
            pltpu.async_copy(zrow.at[pl.ds(bo, F)],
                             out_hbm.at[pl.ds(g * F, F)], otsem[buf])

            tscr[pl.ds(0, 16)] = _vec(sv0[0] - MARGIN)

        def pair_step(i, d):
            g0 = base + 2 * i
            pltpu.async_copy(y_hbm.at[pl.ds((g0 + 1) * F, F)],
                             ybuf.at[pl.ds(F, F)], in1)
            process_row(i, g0, 0)

            @pl.when(i < nrows // 2 - 1)
            def _pref():
                pltpu.async_copy(y_hbm.at[pl.ds((g0 + 2) * F, F)],
                                 ybuf.at[pl.ds(0, F)], in0)

            process_row(i, g0 + 1, 1)
            return d

        lax.fori_loop(0, nrows // 2, pair_step, 0)
        pltpu.make_async_copy(zrow.at[pl.ds(0, F)],
                              out_hbm.at[pl.ds((base + nrows - 2) * F, F)],
                              ot0).wait()
        pltpu.make_async_copy(zrow.at[pl.ds(F, F)],
                              out_hbm.at[pl.ds((base + nrows - 1) * F, F)],
                              ot1).wait()

    return sel(yf)


@jax.jit
def _run(x, W, b):
    B, S, D = x.shape
    F = W.shape[0]
    y = _matmul(x.astype(jnp.bfloat16), W.astype(jnp.bfloat16), b)
    out = _sc_select(y.reshape(S * F), S, F)
    return out.reshape(B, S, F)


def kernel(x, token_mask, W, b):
    del token_mask
    return _run(x, W, b)


# speedup vs baseline: 2.1656x; 1.0279x over previous
import functools

import jax
import jax.numpy as jnp
import numpy as np
from jax import lax
from jax.experimental import pallas as pl
from jax.experimental.pallas import tpu as pltpu
from jax.experimental.pallas import tpu_sc as plsc

K_TOP = 32
FEAT_BLK = 1024
NEG_BIG = np.float32(-3.0e38)
MARGIN = np.float32(0.25)


def _mm_body(x_ref, w_ref, b_ref, y_ref):
    y = lax.dot_general(x_ref[0], w_ref[...], (((1,), (1,)), ((), ())),
                        preferred_element_type=jnp.float32)
    y_ref[...] = y + b_ref[...][None, :]


def _matmul(x, W, b):
    B, S, D = x.shape
    F = W.shape[0]
    grid = (F // FEAT_BLK,)
    return pl.pallas_call(
        _mm_body,
        grid=grid,
        in_specs=[
            pl.BlockSpec((1, S, D), lambda f: (0, 0, 0)),
            pl.BlockSpec((FEAT_BLK, D), lambda f: (f, 0)),
            pl.BlockSpec((FEAT_BLK,), lambda f: (f,)),
        ],
        out_specs=pl.BlockSpec((S, FEAT_BLK), lambda f: (0, f)),
        out_shape=jax.ShapeDtypeStruct((S, F), jnp.float32),
        compiler_params=pltpu.CompilerParams(
            dimension_semantics=("arbitrary",)),
    )(x, W, b)


def _vec(x):
    return lax.broadcast_in_dim(x, (16,), ())


def _merge32(sv0, si0, sv1, si1, nv, ni):
    nvs, nis = plsc.sort_key_val(nv, ni)
    rv = lax.rev(nvs, (0,))
    ri = lax.rev(nis, (0,))
    sel = sv0 >= rv
    hv = jnp.where(sel, sv0, rv)
    hi = jnp.where(sel, si0, ri)
    hvs, his = plsc.sort_key_val(hv, hi)
    rhv = lax.rev(hvs, (0,))
    rhi = lax.rev(his, (0,))
    sel2 = sv1 >= rhv
    new_hi_v = jnp.where(sel2, sv1, rhv)
    new_hi_i = jnp.where(sel2, si1, rhi)
    new_lo_v = jnp.where(sel2, rhv, sv1)
    new_lo_i = jnp.where(sel2, rhi, si1)
    lv, li = plsc.sort_key_val(new_lo_v, new_lo_i)
    uv, ui = plsc.sort_key_val(new_hi_v, new_hi_i)
    return lv, li, uv, ui


def _sc_select(yf, S, F, interpret=False):
    mesh = plsc.VectorSubcoreMesh(core_axis_name="c", subcore_axis_name="s",
                                  num_cores=2, num_subcores=16)
    nrows = S // 32

    @functools.partial(
        pl.kernel,
        out_type=jax.ShapeDtypeStruct((S * F,), jnp.float32),
        mesh=mesh,
        interpret=interpret,
        compiler_params=pltpu.CompilerParams(needs_layout_passes=False),
        scratch_types=[
            pltpu.VMEM((2 * F,), jnp.float32),
            pltpu.VMEM((2 * F,), jnp.float32),
            pltpu.VMEM((F + 16,), jnp.float32),
            pltpu.VMEM((F + 16,), jnp.int32),
            pltpu.VMEM((64,), jnp.int32),
            pltpu.VMEM((16,), jnp.float32),
            pltpu.VMEM((32,), jnp.float32),
            pltpu.VMEM((32,), jnp.int32),
            pltpu.SemaphoreType.DMA,
            pltpu.SemaphoreType.DMA,
            pltpu.SemaphoreType.DMA,
            pltpu.SemaphoreType.DMA,
        ],
    )
    def sel(y_hbm, out_hbm, ybuf, zrow, cv, ci, sidx, tscr, stv, sti,
            in0, in1, ot0, ot1):
        wid = lax.axis_index("s") * 2 + lax.axis_index("c")
        base = wid * nrows
        iota = lax.iota(jnp.int32, 16)
        zeros16 = jnp.zeros((16,), jnp.float32)
        insem = (in0, in1)
        otsem = (ot0, ot1)

        def _z(i, _):
            zrow[pl.ds(i * 16, 16)] = zeros16
            zrow[pl.ds(F + i * 16, 16)] = zeros16
            return _

        lax.fori_loop(0, F // 16, _z, 0)
        tscr[pl.ds(0, 16)] = _vec(NEG_BIG)

        pltpu.async_copy(y_hbm.at[pl.ds(base * F, F)],
                         ybuf.at[pl.ds(0, F)], in0)

        def popcnt(m):
            return plsc.all_reduce_population_count(m)[0]

        def collect(bo, tcv, iv):
            def subgrp(offs, anyc, L):
                def ghit(Lg):
                    ms = []
                    cvec = jnp.zeros((16,), jnp.int32)
                    for j in range(8):
                        v = ybuf[pl.ds(bo + offs + j * 16, 16)]
                        m = v >= tcv
                        ms.append((v, m))
                        pc = plsc.all_reduce_population_count(m)
                        cvec = jnp.where(iota == j, pc, cvec)
                    pref = plsc.cumsum(cvec)
                    excl = pref - cvec
                    for j in range(8):
                        v, m = ms[j]
                        o = Lg + excl[j]
                        plsc.store_compressed(cv.at[pl.ds(o, 16)], v,
                                              mask=m)
                        plsc.store_compressed(
                            ci.at[pl.ds(o, 16)],
                            iota + (offs + j * 16), mask=m)
                    return Lg + pref[15]

                return lax.cond(anyc > 0, ghit, lambda Lg: Lg, L)

            def grp(gg, L):
                offs0 = gg * 256
                offs1 = gg * 256 + 128
                om0 = ybuf[pl.ds(bo + offs0, 16)] >= tcv
                om1 = ybuf[pl.ds(bo + offs1, 16)] >= tcv
                for j in range(1, 8):
                    om0 = jnp.logical_or(
                        om0, ybuf[pl.ds(bo + offs0 + j * 16, 16)] >= tcv)
                    om1 = jnp.logical_or(
                        om1, ybuf[pl.ds(bo + offs1 + j * 16, 16)] >= tcv)
                a0 = plsc.all_reduce_population_count(om0)
                a1 = plsc.all_reduce_population_count(om1)
                L = subgrp(offs0, a0[0], L)
                L = subgrp(offs1, a1[0], L)
                return L

            return lax.fori_loop(0, F // 256, grp, iv)

        def process_row(i, g, buf):
            bo = buf * F
            pltpu.make_async_copy(y_hbm.at[pl.ds(g * F, F)],
                                  ybuf.at[pl.ds(bo, F)],
                                  insem[buf]).wait()

            @pl.when(i == 0)
            def _boot():
                if buf == 0:
                    def mx(t, _):
                        mv = jnp.maximum(ybuf[pl.ds(t * 64, 16)],
                                         ybuf[pl.ds(t * 64 + 16, 16)])
                        mv2 = jnp.maximum(ybuf[pl.ds(t * 64 + 32, 16)],
                                          ybuf[pl.ds(t * 64 + 48, 16)])
                        cur = tscr[pl.ds(0, 16)]
                        tscr[pl.ds(0, 16)] = jnp.maximum(
                            cur, jnp.maximum(mv, mv2))
                        return _

                    lax.fori_loop(0, F // 64, mx, 0)
                    m16 = tscr[pl.ds(0, 16)]
                    sm, _d = plsc.sort_key_val(m16, iota)
                    tscr[pl.ds(0, 16)] = _vec(sm[15] * np.float32(0.6))

            tcv = tscr[pl.ds(0, 16)]
            L = collect(bo, tcv, 0)
            L = lax.cond(L < K_TOP,
                         lambda _: collect(bo, _vec(NEG_BIG), 0),
                         lambda Lk: Lk, L)
            cv[pl.ds(L, 16)] = _vec(NEG_BIG)
            ci[pl.ds(L, 16)] = iota

            stv[pl.ds(0, 16)] = _vec(NEG_BIG)
            stv[pl.ds(16, 16)] = _vec(NEG_BIG)
            sti[pl.ds(0, 16)] = iota
            sti[pl.ds(16, 16)] = iota + 16
            nvec = (L + 15) // 16

            def mstep(t, d):
                nv = cv[pl.ds(t * 16, 16)]
                ni = ci[pl.ds(t * 16, 16)]
                a, b_, c_, d_ = _merge32(
                    stv[pl.ds(0, 16)], sti[pl.ds(0, 16)],
                    stv[pl.ds(16, 16)], sti[pl.ds(16, 16)], nv, ni)
                stv[pl.ds(0, 16)] = a
                sti[pl.ds(0, 16)] = b_
                stv[pl.ds(16, 16)] = c_
                sti[pl.ds(16, 16)] = d_
                return d

            lax.fori_loop(0, nvec, mstep, 0)
            sv0 = stv[pl.ds(0, 16)]
            sv1 = stv[pl.ds(16, 16)]
            si0 = sti[pl.ds(0, 16)] + bo
            si1 = sti[pl.ds(16, 16)] + bo

            @pl.when(i >= 1)
            def _drain():
                pltpu.make_async_copy(zrow.at[pl.ds(bo, F)],
                                      out_hbm.at[pl.ds((g - 2) * F, F)],
                                      otsem[buf]).wait()
                oi0 = sidx[pl.ds(buf * 32, 16)]
                oi1 = sidx[pl.ds(buf * 32 + 16, 16)]
                plsc.store_scatter(zrow, [oi0], zeros16)
                plsc.store_scatter(zrow, [oi1], zeros16)

            plsc.store_scatter(zrow, [si0], jnp.maximum(sv0, 0.0))
            plsc.store_scatter(zrow, [si1], jnp.maximum(sv1, 0.0))
            sidx[pl.ds(buf * 32, 16)] = si0
            sidx[pl.ds(buf * 32 + 16, 16)] = si1
            pltpu.async_copy(zrow.at[pl.ds(bo, F)],
                             out_hbm.at[pl.ds(g * F, F)], otsem[buf])

            tscr[pl.ds(0, 16)] = _vec(sv0[0] - MARGIN)

        def pair_step(i, d):
            g0 = base + 2 * i
            pltpu.async_copy(y_hbm.at[pl.ds((g0 + 1) * F, F)],
                             ybuf.at[pl.ds(F, F)], in1)
            process_row(i, g0, 0)

            @pl.when(i < nrows // 2 - 1)
            def _pref():
                pltpu.async_copy(y_hbm.at[pl.ds((g0 + 2) * F, F)],
                                 ybuf.at[pl.ds(0, F)], in0)

            process_row(i, g0 + 1, 1)
            return d

        lax.fori_loop(0, nrows // 2, pair_step, 0)
        pltpu.make_async_copy(zrow.at[pl.ds(0, F)],
                              out_hbm.at[pl.ds((base + nrows - 2) * F, F)],
                              ot0).wait()
        pltpu.make_async_copy(zrow.at[pl.ds(F, F)],
                              out_hbm.at[pl.ds((base + nrows - 1) * F, F)],
                              ot1).wait()

    return sel(yf)


@jax.jit
def _run(x, W, b):
    B, S, D = x.shape
    F = W.shape[0]
    y = _matmul(x.astype(jnp.bfloat16), W.astype(jnp.bfloat16), b)
    out = _sc_select(y.reshape(S * F), S, F)
    return out.reshape(B, S, F)


def kernel(x, token_mask, W, b):
    del token_mask
    return _run(x, W, b)


# speedup vs baseline: 2.1660x; 1.0002x over previous
import functools

import jax
import jax.numpy as jnp
import numpy as np
from jax import lax
from jax.experimental import pallas as pl
from jax.experimental.pallas import tpu as pltpu
from jax.experimental.pallas import tpu_sc as plsc

K_TOP = 32
FEAT_BLK = 1024
NEG_BIG = np.float32(-3.0e38)
MARGIN = np.float32(0.25)


def _mm_body(x_ref, w_ref, b_ref, y_ref):
    y = lax.dot_general(x_ref[0], w_ref[...], (((1,), (1,)), ((), ())),
                        preferred_element_type=jnp.float32)
    y_ref[...] = y + b_ref[...][None, :]


def _matmul(x, W, b):
    B, S, D = x.shape
    F = W.shape[0]
    grid = (F // FEAT_BLK,)
    return pl.pallas_call(
        _mm_body,
        grid=grid,
        in_specs=[
            pl.BlockSpec((1, S, D), lambda f: (0, 0, 0)),
            pl.BlockSpec((FEAT_BLK, D), lambda f: (f, 0)),
            pl.BlockSpec((FEAT_BLK,), lambda f: (f,)),
        ],
        out_specs=pl.BlockSpec((S, FEAT_BLK), lambda f: (0, f)),
        out_shape=jax.ShapeDtypeStruct((S, F), jnp.float32),
        compiler_params=pltpu.CompilerParams(
            dimension_semantics=("arbitrary",)),
    )(x, W, b)


def _vec(x):
    return lax.broadcast_in_dim(x, (16,), ())


def _merge32(sv0, si0, sv1, si1, nv, ni):
    nvs, nis = plsc.sort_key_val(nv, ni)
    rv = lax.rev(nvs, (0,))
    ri = lax.rev(nis, (0,))
    sel = sv0 >= rv
    hv = jnp.where(sel, sv0, rv)
    hi = jnp.where(sel, si0, ri)
    hvs, his = plsc.sort_key_val(hv, hi)
    rhv = lax.rev(hvs, (0,))
    rhi = lax.rev(his, (0,))
    sel2 = sv1 >= rhv
    new_hi_v = jnp.where(sel2, sv1, rhv)
    new_hi_i = jnp.where(sel2, si1, rhi)
    new_lo_v = jnp.where(sel2, rhv, sv1)
    new_lo_i = jnp.where(sel2, rhi, si1)
    lv, li = plsc.sort_key_val(new_lo_v, new_lo_i)
    uv, ui = plsc.sort_key_val(new_hi_v, new_hi_i)
    return lv, li, uv, ui


def _sc_select(yf, S, F):
    mesh = plsc.VectorSubcoreMesh(core_axis_name="c", subcore_axis_name="s",
                                  num_cores=2, num_subcores=16)
    nrows = S // 32

    @functools.partial(
        pl.kernel,
        out_type=jax.ShapeDtypeStruct((S * F,), jnp.float32),
        mesh=mesh,
        compiler_params=pltpu.CompilerParams(needs_layout_passes=False),
        scratch_types=[
            pltpu.VMEM((2 * F,), jnp.float32),
            pltpu.VMEM((2 * F,), jnp.float32),
            pltpu.VMEM((F + 16,), jnp.float32),
            pltpu.VMEM((F + 16,), jnp.int32),
            pltpu.VMEM((64,), jnp.int32),
            pltpu.VMEM((16,), jnp.float32),
            pltpu.VMEM((32,), jnp.float32),
            pltpu.VMEM((32,), jnp.int32),
            pltpu.SemaphoreType.DMA,
            pltpu.SemaphoreType.DMA,
            pltpu.SemaphoreType.DMA,
            pltpu.SemaphoreType.DMA,
        ],
    )
    def sel(y_hbm, out_hbm, ybuf, zrow, cv, ci, sidx, tscr, stv, sti,
            in0, in1, ot0, ot1):
        wid = lax.axis_index("s") * 2 + lax.axis_index("c")
        base = wid * nrows
        iota = lax.iota(jnp.int32, 16)
        zeros16 = jnp.zeros((16,), jnp.float32)
        insem = (in0, in1)
        otsem = (ot0, ot1)

        def _z(i, _):
            zrow[pl.ds(i * 16, 16)] = zeros16
            zrow[pl.ds(F + i * 16, 16)] = zeros16
            return _

        lax.fori_loop(0, F // 16, _z, 0)
        tscr[pl.ds(0, 16)] = _vec(NEG_BIG)

        pltpu.async_copy(y_hbm.at[pl.ds(base * F, F)],
                         ybuf.at[pl.ds(0, F)], in0)

        def popcnt(m):
            return plsc.all_reduce_population_count(m)[0]

        def collect(bo, tcv, iv):
            def subgrp(offs, anyc, L):
                def ghit(Lg):
                    ms = []
                    cvec = jnp.zeros((16,), jnp.int32)
                    for j in range(8):
                        v = ybuf[pl.ds(bo + offs + j * 16, 16)]
                        m = v >= tcv
                        ms.append((v, m))
                        pc = plsc.all_reduce_population_count(m)
                        cvec = jnp.where(iota == j, pc, cvec)
                    pref = plsc.cumsum(cvec)
                    excl = pref - cvec
                    for j in range(8):
                        v, m = ms[j]
                        o = Lg + excl[j]
                        plsc.store_compressed(cv.at[pl.ds(o, 16)], v,
                                              mask=m)
                        plsc.store_compressed(
                            ci.at[pl.ds(o, 16)],
                            iota + (offs + j * 16), mask=m)
                    return Lg + pref[15]

                return lax.cond(anyc > 0, ghit, lambda Lg: Lg, L)

            def grp(gg, L):
                offs0 = gg * 256
                offs1 = gg * 256 + 128
                om0 = ybuf[pl.ds(bo + offs0, 16)] >= tcv
                om1 = ybuf[pl.ds(bo + offs1, 16)] >= tcv
                for j in range(1, 8):
                    om0 = jnp.logical_or(
                        om0, ybuf[pl.ds(bo + offs0 + j * 16, 16)] >= tcv)
                    om1 = jnp.logical_or(
                        om1, ybuf[pl.ds(bo + offs1 + j * 16, 16)] >= tcv)
                a0 = plsc.all_reduce_population_count(om0)
                a1 = plsc.all_reduce_population_count(om1)
                L = subgrp(offs0, a0[0], L)
                L = subgrp(offs1, a1[0], L)
                return L

            return lax.fori_loop(0, F // 256, grp, iv)

        def process_row(i, g, buf):
            bo = buf * F
            pltpu.make_async_copy(y_hbm.at[pl.ds(g * F, F)],
                                  ybuf.at[pl.ds(bo, F)],
                                  insem[buf]).wait()

            @pl.when(i == 0)
            def _boot():
                if buf == 0:
                    def mx(t, _):
                        mv = jnp.maximum(ybuf[pl.ds(t * 64, 16)],
                                         ybuf[pl.ds(t * 64 + 16, 16)])
                        mv2 = jnp.maximum(ybuf[pl.ds(t * 64 + 32, 16)],
                                          ybuf[pl.ds(t * 64 + 48, 16)])
                        cur = tscr[pl.ds(0, 16)]
                        tscr[pl.ds(0, 16)] = jnp.maximum(
                            cur, jnp.maximum(mv, mv2))
                        return _

                    lax.fori_loop(0, F // 64, mx, 0)
                    m16 = tscr[pl.ds(0, 16)]
                    sm, _d = plsc.sort_key_val(m16, iota)
                    tscr[pl.ds(0, 16)] = _vec(sm[15] * np.float32(0.6))

            tcv = tscr[pl.ds(0, 16)]
            L = collect(bo, tcv, 0)
            L = lax.cond(L < K_TOP,
                         lambda _: collect(bo, _vec(NEG_BIG), 0),
                         lambda Lk: Lk, L)
            cv[pl.ds(L, 16)] = _vec(NEG_BIG)
            ci[pl.ds(L, 16)] = iota

            stv[pl.ds(0, 16)] = _vec(NEG_BIG)
            stv[pl.ds(16, 16)] = _vec(NEG_BIG)
            sti[pl.ds(0, 16)] = iota
            sti[pl.ds(16, 16)] = iota + 16
            nvec = (L + 15) // 16

            def mstep(t, d):
                nv = cv[pl.ds(t * 16, 16)]
                ni = ci[pl.ds(t * 16, 16)]
                a, b_, c_, d_ = _merge32(
                    stv[pl.ds(0, 16)], sti[pl.ds(0, 16)],
                    stv[pl.ds(16, 16)], sti[pl.ds(16, 16)], nv, ni)
                stv[pl.ds(0, 16)] = a
                sti[pl.ds(0, 16)] = b_
                stv[pl.ds(16, 16)] = c_
                sti[pl.ds(16, 16)] = d_
                return d

            lax.fori_loop(0, nvec, mstep, 0)
            sv0 = stv[pl.ds(0, 16)]
            sv1 = stv[pl.ds(16, 16)]
            si0 = sti[pl.ds(0, 16)] + bo
            si1 = sti[pl.ds(16, 16)] + bo

            @pl.when(i >= 1)
            def _drain():
                pltpu.make_async_copy(zrow.at[pl.ds(bo, F)],
                                      out_hbm.at[pl.ds((g - 2) * F, F)],
                                      otsem[buf]).wait()
                oi0 = sidx[pl.ds(buf * 32, 16)]
                oi1 = sidx[pl.ds(buf * 32 + 16, 16)]
                plsc.store_scatter(zrow, [oi0], zeros16)
                plsc.store_scatter(zrow, [oi1], zeros16)

            plsc.store_scatter(zrow, [si0], jnp.maximum(sv0, 0.0))
            plsc.store_scatter(zrow, [si1], jnp.maximum(sv1, 0.0))
            sidx[pl.ds(buf * 32, 16)] = si0
            sidx[pl.ds(buf * 32 + 16, 16)] = si1
            pltpu.async_copy(zrow.at[pl.ds(bo, F)],
                             out_hbm.at[pl.ds(g * F, F)], otsem[buf])

            tscr[pl.ds(0, 16)] = _vec(sv0[0] - MARGIN)

        def pair_step(i, d):
            g0 = base + 2 * i
            pltpu.async_copy(y_hbm.at[pl.ds((g0 + 1) * F, F)],
                             ybuf.at[pl.ds(F, F)], in1)
            process_row(i, g0, 0)

            @pl.when(i < nrows // 2 - 1)
            def _pref():
                pltpu.async_copy(y_hbm.at[pl.ds((g0 + 2) * F, F)],
                                 ybuf.at[pl.ds(0, F)], in0)

            process_row(i, g0 + 1, 1)
            return d

        lax.fori_loop(0, nrows // 2, pair_step, 0)
        pltpu.make_async_copy(zrow.at[pl.ds(0, F)],
                              out_hbm.at[pl.ds((base + nrows - 2) * F, F)],
                              ot0).wait()
        pltpu.make_async_copy(zrow.at[pl.ds(F, F)],
                              out_hbm.at[pl.ds((base + nrows - 1) * F, F)],
                              ot1).wait()

    return sel(yf)


@jax.jit
def _run(x, W, b):
    B, S, D = x.shape
    F = W.shape[0]
    y = _matmul(x.astype(jnp.bfloat16), W.astype(jnp.bfloat16), b)
    out = _sc_select(y.reshape(S * F), S, F)
    return out.reshape(B, S, F)


def kernel(x, token_mask, W, b):
    del token_mask
    return _run(x, W, b)
